# Initial kernel scaffold; baseline (speedup 1.0000x reference)
import functools

import jax
import jax.numpy as jnp
from jax import lax
from jax.experimental import pallas as pl
from jax.experimental.pallas import tpu as pltpu
from jax.experimental.pallas import tpu_sc as plsc

N = 10000
E = 320000
C_IN = 128
C_EDGE = 17
H = 4
C_KQ = 64
C_V = 64
C_OUT = 128
R_HID = 32

NC = 2
NS = 16
NW = NC * NS
LP = 16
CHUNK = 80
E_PER_TILE = E // NW
N_CHUNKS = E_PER_TILE // CHUNK
N_PER_TILE = N // NS
ACC_W = 80


def _node_pre_body(x0_ref, wkv_ref, wq_ref, wnx_ref,
                   xkvk_ref, xkvv_ref, qs_ref, x0wn_ref):
    x = x0_ref[...]
    kv = jnp.dot(x, wkv_ref[...], preferred_element_type=jnp.float32)
    xkvv_ref[...] = kv[:, :C_V]
    xkvk_ref[...] = kv[:, C_V:]
    qs_ref[...] = jnp.dot(x, wq_ref[...], preferred_element_type=jnp.float32) * 0.125
    x0wn_ref[...] = jnp.dot(x, wnx_ref[...], preferred_element_type=jnp.float32)


def _edge_pre_body(ef_ref, wr1_ref, br1_ref, wr2_ref, wee_ref,
                   radk_ref, radv_ref, ebase_ref):
    ef = ef_ref[...]
    h = jnp.maximum(jnp.dot(ef, wr1_ref[...], preferred_element_type=jnp.float32)
                    + br1_ref[...], 0.0)
    rad = jnp.dot(h, wr2_ref[...], preferred_element_type=jnp.float32)
    radv_ref[...] = rad[:, :C_V]
    radk_ref[...] = rad[:, C_V:]
    ebase_ref[...] = jnp.dot(ef, wee_ref[...], preferred_element_type=jnp.float32)


def _edge_out_body(ebase_ref, lg_ref, wel_ref, eout_ref):
    lg = lg_ref[...][:, :H]
    eout_ref[...] = ebase_ref[...] + jnp.dot(
        lg, wel_ref[...], preferred_element_type=jnp.float32)


def _node_out_body(u2_ref, x0wn_ref, wnz_ref, nout_ref):
    u = u2_ref[0] + u2_ref[1]
    w = u[:, :C_V]
    s4 = u[:, C_V:C_V + H]
    hh = lax.broadcasted_iota(jnp.int32, (H, C_V), 0)
    ll = lax.broadcasted_iota(jnp.int32, (H, C_V), 1) // (C_V // H)
    rep = (hh == ll).astype(jnp.float32)
    srep = jnp.dot(s4, rep, preferred_element_type=jnp.float32)
    z = w / jnp.maximum(srep, 1e-30)
    nout_ref[...] = jnp.dot(z, wnz_ref[...], preferred_element_type=jnp.float32) \
        + x0wn_ref[...]


def _sc_logits_body(src_hbm, dst_hbm, xkvk_hbm, qs_hbm, radk_hbm,
                    lg_hbm, tmax_hbm,
                    idxs_v, idxd_v, xk_v, q_v, rk_v, lg_v, m_v, sem):
    cid = lax.axis_index("c")
    sid = lax.axis_index("s")
    wid = sid * NC + cid
    tile_base = wid * E_PER_TILE

    def chunk_body(j, m_carry):
        base = tile_base + j * CHUNK
        pltpu.sync_copy(src_hbm.at[pl.ds(base, CHUNK)], idxs_v)
        pltpu.sync_copy(dst_hbm.at[pl.ds(base, CHUNK)], idxd_v)
        pltpu.sync_copy(radk_hbm.at[pl.ds(base, CHUNK), :], rk_v)
        pltpu.async_copy(xkvk_hbm.at[idxs_v], xk_v, sem).wait()
        pltpu.async_copy(qs_hbm.at[idxd_v], q_v, sem).wait()

        def edge_body(e, m_in):
            m_out = m_in
            for h in range(H):
                a = xk_v[e, pl.ds(h * LP, LP)]
                b = rk_v[e, pl.ds(h * LP, LP)]
                c = q_v[e, pl.ds(h * LP, LP)]
                s = jnp.sum(a * b * c)
                lg_v[e, h] = s
                m_out = jnp.maximum(m_out, s)
            return m_out

        m2 = lax.fori_loop(0, CHUNK, edge_body, m_carry)
        pltpu.sync_copy(lg_v, lg_hbm.at[pl.ds(base, CHUNK), :])
        return m2

    m = lax.fori_loop(0, N_CHUNKS, chunk_body, jnp.float32(-3.0e38))
    m_v[...] = jnp.full((LP,), m, dtype=jnp.float32)
    pltpu.sync_copy(m_v, tmax_hbm.at[wid])


def _sc_scatter_body(src_hbm, dst_hbm, lg_hbm, xkvv_hbm, radv_hbm,
                     tmax_hbm, zeros_hbm,
                     u_hbm,
                     idxs_v, idxd_v, xv_v, rv_v, lg_v, w_v, tm_v, acc_sh, sem):
    cid = lax.axis_index("c")
    sid = lax.axis_index("s")
    wid = sid * NC + cid
    tile_base = wid * E_PER_TILE

    pltpu.sync_copy(tmax_hbm, tm_v)

    def max_body(i, m_in):
        return jnp.maximum(m_in, jnp.max(tm_v[i]))

    gm = lax.fori_loop(0, NW, max_body, jnp.float32(-3.0e38))

    pltpu.sync_copy(zeros_hbm.at[pl.ds(sid * N_PER_TILE, N_PER_TILE), :],
                    acc_sh.at[pl.ds(sid * N_PER_TILE, N_PER_TILE), :])
    plsc.subcore_barrier()

    lane = lax.iota(jnp.int32, LP)

    def chunk_body(j, carry):
        base = tile_base + j * CHUNK
        pltpu.sync_copy(src_hbm.at[pl.ds(base, CHUNK)], idxs_v)
        pltpu.sync_copy(dst_hbm.at[pl.ds(base, CHUNK)], idxd_v)
        pltpu.sync_copy(radv_hbm.at[pl.ds(base, CHUNK), :], rv_v)
        pltpu.sync_copy(lg_hbm.at[pl.ds(base, CHUNK), :], lg_v)
        pltpu.async_copy(xkvv_hbm.at[idxs_v], xv_v, sem).wait()

        def edge_body(e, c2):
            lrow = lg_v[e, :]
            ex = jnp.exp(lrow - gm)
            ex = jnp.where(lane < H, ex, 0.0)
            w_v[e, pl.ds(C_V, LP)] = ex
            for h in range(H):
                ex_s = w_v[e, C_V + h]
                xv = xv_v[e, pl.ds(h * LP, LP)]
                rv = rv_v[e, pl.ds(h * LP, LP)]
                w_v[e, pl.ds(h * LP, LP)] = xv * rv * ex_s
            return c2

        lax.fori_loop(0, CHUNK, edge_body, 0)
        pltpu.sync_copy(w_v, acc_sh.at[idxd_v], add=True)
        return carry

    lax.fori_loop(0, N_CHUNKS, chunk_body, 0)
    plsc.subcore_barrier()
    pltpu.sync_copy(acc_sh.at[pl.ds(sid * N_PER_TILE, N_PER_TILE), :],
                    u_hbm.at[cid, pl.ds(sid * N_PER_TILE, N_PER_TILE), :])


def kernel(x0, edge_feat, edge_index, W_r1, b_r1, W_r2, W_kv, W_q, W_node,
           W_edge):
    f32 = jnp.float32
    x0_2d = x0[:, :, 0]
    ef = edge_feat[:, :, 0]
    src = edge_index[0]
    dst = edge_index[1]
    b_r1_2d = b_r1[None, :]
    W_node_z = W_node[:C_V]
    W_node_x = W_node[C_V:]
    W_edge_e = W_edge[:C_EDGE]
    W_edge_l = W_edge[C_EDGE:]

    NB = 1000
    xkv_k, xkv_v, qs, x0wn = pl.pallas_call(
        _node_pre_body,
        grid=(N // NB,),
        in_specs=[
            pl.BlockSpec((NB, C_IN), lambda i: (i, 0)),
            pl.BlockSpec((C_IN, C_V + C_KQ), lambda i: (0, 0)),
            pl.BlockSpec((C_IN, C_KQ), lambda i: (0, 0)),
            pl.BlockSpec((C_IN, C_OUT), lambda i: (0, 0)),
        ],
        out_specs=[
            pl.BlockSpec((NB, C_KQ), lambda i: (i, 0)),
            pl.BlockSpec((NB, C_V), lambda i: (i, 0)),
            pl.BlockSpec((NB, C_KQ), lambda i: (i, 0)),
            pl.BlockSpec((NB, C_OUT), lambda i: (i, 0)),
        ],
        out_shape=[
            jax.ShapeDtypeStruct((N, C_KQ), f32),
            jax.ShapeDtypeStruct((N, C_V), f32),
            jax.ShapeDtypeStruct((N, C_KQ), f32),
            jax.ShapeDtypeStruct((N, C_OUT), f32),
        ],
    )(x0_2d, W_kv, W_q, W_node_x)

    EB = 4000
    rad_k, rad_v, ebase = pl.pallas_call(
        _edge_pre_body,
        grid=(E // EB,),
        in_specs=[
            pl.BlockSpec((EB, C_EDGE), lambda i: (i, 0)),
            pl.BlockSpec((C_EDGE, R_HID), lambda i: (0, 0)),
            pl.BlockSpec((1, R_HID), lambda i: (0, 0)),
            pl.BlockSpec((R_HID, C_V + C_KQ), lambda i: (0, 0)),
            pl.BlockSpec((C_EDGE, C_EDGE), lambda i: (0, 0)),
        ],
        out_specs=[
            pl.BlockSpec((EB, C_KQ), lambda i: (i, 0)),
            pl.BlockSpec((EB, C_V), lambda i: (i, 0)),
            pl.BlockSpec((EB, C_EDGE), lambda i: (i, 0)),
        ],
        out_shape=[
            jax.ShapeDtypeStruct((E, C_KQ), f32),
            jax.ShapeDtypeStruct((E, C_V), f32),
            jax.ShapeDtypeStruct((E, C_EDGE), f32),
        ],
    )(ef, W_r1, b_r1_2d, W_r2, W_edge_e)

    mesh = plsc.VectorSubcoreMesh(core_axis_name="c", subcore_axis_name="s")

    sc_a = pl.kernel(
        _sc_logits_body,
        out_type=(
            jax.ShapeDtypeStruct((E, LP), f32),
            jax.ShapeDtypeStruct((NW, LP), f32),
        ),
        mesh=mesh,
        scratch_types=[
            pltpu.VMEM((CHUNK,), jnp.int32),
            pltpu.VMEM((CHUNK,), jnp.int32),
            pltpu.VMEM((CHUNK, C_KQ), f32),
            pltpu.VMEM((CHUNK, C_KQ), f32),
            pltpu.VMEM((CHUNK, C_KQ), f32),
            pltpu.VMEM((CHUNK, LP), f32),
            pltpu.VMEM((LP,), f32),
            pltpu.SemaphoreType.DMA,
        ],
    )
    logits16, tmax = sc_a(src, dst, xkv_k, qs, rad_k)

    zeros_acc = jnp.zeros((N, ACC_W), f32)
    sc_b = pl.kernel(
        _sc_scatter_body,
        out_type=jax.ShapeDtypeStruct((NC, N, ACC_W), f32),
        mesh=mesh,
        scratch_types=[
            pltpu.VMEM((CHUNK,), jnp.int32),
            pltpu.VMEM((CHUNK,), jnp.int32),
            pltpu.VMEM((CHUNK, C_V), f32),
            pltpu.VMEM((CHUNK, C_V), f32),
            pltpu.VMEM((CHUNK, LP), f32),
            pltpu.VMEM((CHUNK, ACC_W), f32),
            pltpu.VMEM((NW, LP), f32),
            pltpu.VMEM_SHARED((N, ACC_W), f32),
            pltpu.SemaphoreType.DMA,
        ],
    )
    u2 = sc_b(src, dst, logits16, xkv_v, rad_v, tmax, zeros_acc)

    node_out = pl.pallas_call(
        _node_out_body,
        grid=(N // NB,),
        in_specs=[
            pl.BlockSpec((NC, NB, ACC_W), lambda i: (0, i, 0)),
            pl.BlockSpec((NB, C_OUT), lambda i: (i, 0)),
            pl.BlockSpec((C_V, C_OUT), lambda i: (0, 0)),
        ],
        out_specs=pl.BlockSpec((NB, C_OUT), lambda i: (i, 0)),
        out_shape=jax.ShapeDtypeStruct((N, C_OUT), f32),
    )(u2, x0wn, W_node_z)

    edge_out = pl.pallas_call(
        _edge_out_body,
        grid=(E // EB,),
        in_specs=[
            pl.BlockSpec((EB, C_EDGE), lambda i: (i, 0)),
            pl.BlockSpec((EB, LP), lambda i: (i, 0)),
            pl.BlockSpec((H, C_EDGE), lambda i: (0, 0)),
        ],
        out_specs=pl.BlockSpec((EB, C_EDGE), lambda i: (i, 0)),
        out_shape=jax.ShapeDtypeStruct((E, C_EDGE), f32),
    )(ebase, logits16, W_edge_l)

    return (node_out[:, :, None], edge_out[:, :, None])


# speedup vs baseline: 19.0434x; 19.0434x over previous
import functools

import jax
import jax.numpy as jnp
from jax import lax
from jax.experimental import pallas as pl
from jax.experimental.pallas import tpu as pltpu
from jax.experimental.pallas import tpu_sc as plsc

N = 10000
E = 320000
C_IN = 128
C_EDGE = 17
H = 4
C_KQ = 64
C_V = 64
C_OUT = 128
R_HID = 32

NC = 2
NS = 16
NW = NC * NS
LP = 16
CHUNK = 80
E_PER_TILE = E // NW
N_CHUNKS = E_PER_TILE // CHUNK
N_PER_TILE = N // NS
ACC_W = 80


def _node_pre_body(x0_ref, wkv_ref, wq_ref, wnx_ref,
                   xkvk_ref, xkvv_ref, qs_ref, x0wn_ref):
    x = x0_ref[...]
    kv = jnp.dot(x, wkv_ref[...], preferred_element_type=jnp.float32)
    xkvv_ref[...] = kv[:, :C_V]
    xkvk_ref[...] = kv[:, C_V:]
    qs_ref[...] = jnp.dot(x, wq_ref[...], preferred_element_type=jnp.float32) * 0.125
    x0wn_ref[...] = jnp.dot(x, wnx_ref[...], preferred_element_type=jnp.float32)


def _edge_pre_body(ef_ref, wr1_ref, br1_ref, wr2_ref, wee_ref,
                   radk_ref, radv_ref, ebase_ref):
    ef = ef_ref[...]
    h = jnp.maximum(jnp.dot(ef, wr1_ref[...], preferred_element_type=jnp.float32)
                    + br1_ref[...], 0.0)
    rad = jnp.dot(h, wr2_ref[...], preferred_element_type=jnp.float32)
    radv_ref[...] = rad[:, :C_V]
    radk_ref[...] = rad[:, C_V:]
    ebase_ref[...] = jnp.dot(ef, wee_ref[...], preferred_element_type=jnp.float32)


def _edge_out_body(ebase_ref, lg_ref, wel_ref, eout_ref):
    lg = lg_ref[...][:, :H]
    eout_ref[...] = ebase_ref[...] + jnp.dot(
        lg, wel_ref[...], preferred_element_type=jnp.float32)


def _node_out_body(u2_ref, x0wn_ref, wnz_ref, nout_ref):
    u = u2_ref[0] + u2_ref[1]
    w = u[:, :C_V]
    s4 = u[:, C_V:C_V + H]
    hh = lax.broadcasted_iota(jnp.int32, (H, C_V), 0)
    ll = lax.broadcasted_iota(jnp.int32, (H, C_V), 1) // (C_V // H)
    rep = (hh == ll).astype(jnp.float32)
    srep = jnp.dot(s4, rep, preferred_element_type=jnp.float32)
    z = w / jnp.maximum(srep, 1e-30)
    nout_ref[...] = jnp.dot(z, wnz_ref[...], preferred_element_type=jnp.float32) \
        + x0wn_ref[...]


def _sc_logits_body(src_hbm, dst_hbm, xkvk_hbm, qs_hbm, radk_hbm,
                    lg_hbm, tmax_hbm,
                    idxs_v, idxd_v, xk_v, q_v, rk_v, lg_v, m_v, sem):
    cid = lax.axis_index("c")
    sid = lax.axis_index("s")
    wid = sid * NC + cid
    tile_base = wid * E_PER_TILE

    lane = lax.iota(jnp.int32, LP)

    def chunk_body(j, m_carry):
        base = tile_base + j * CHUNK
        pltpu.sync_copy(src_hbm.at[pl.ds(base, CHUNK)], idxs_v)
        pltpu.sync_copy(dst_hbm.at[pl.ds(base, CHUNK)], idxd_v)
        pltpu.sync_copy(radk_hbm.at[pl.ds(base, CHUNK), :], rk_v)
        pltpu.async_copy(xkvk_hbm.at[idxs_v], xk_v, sem).wait()
        pltpu.async_copy(qs_hbm.at[idxd_v], q_v, sem).wait()

        def edge_body(e, m_in):
            m_out = m_in
            srow = jnp.zeros((LP,), jnp.float32)
            for h in range(H):
                a = xk_v[e, pl.ds(h * LP, LP)]
                b = rk_v[e, pl.ds(h * LP, LP)]
                c = q_v[e, pl.ds(h * LP, LP)]
                s = jnp.sum(a * b * c)
                srow = jnp.where(lane == h, s, srow)
                m_out = jnp.maximum(m_out, s)
            lg_v[e, :] = srow
            return m_out

        m2 = lax.fori_loop(0, CHUNK, edge_body, m_carry)
        pltpu.sync_copy(lg_v, lg_hbm.at[pl.ds(base, CHUNK), :])
        return m2

    m = lax.fori_loop(0, N_CHUNKS, chunk_body, jnp.float32(-3.0e38))
    m_v[...] = jnp.full((LP,), m, dtype=jnp.float32)
    pltpu.sync_copy(m_v, tmax_hbm.at[wid])


def _sc_scatter_body(src_hbm, dst_hbm, lg_hbm, xkvv_hbm, radv_hbm,
                     tmax_hbm, zeros_hbm,
                     u_hbm,
                     idxs_v, idxd_v, xv_v, rv_v, lg_v, w_v, tm_v, acc_sh, sem):
    cid = lax.axis_index("c")
    sid = lax.axis_index("s")
    wid = sid * NC + cid
    tile_base = wid * E_PER_TILE

    pltpu.sync_copy(tmax_hbm, tm_v)

    def max_body(i, m_in):
        return jnp.maximum(m_in, jnp.max(tm_v[i]))

    gm = lax.fori_loop(0, NW, max_body, jnp.float32(-3.0e38))

    pltpu.sync_copy(zeros_hbm.at[pl.ds(sid * N_PER_TILE, N_PER_TILE), :],
                    acc_sh.at[pl.ds(sid * N_PER_TILE, N_PER_TILE), :])
    plsc.subcore_barrier()

    lane = lax.iota(jnp.int32, LP)

    def chunk_body(j, carry):
        base = tile_base + j * CHUNK
        pltpu.sync_copy(src_hbm.at[pl.ds(base, CHUNK)], idxs_v)
        pltpu.sync_copy(dst_hbm.at[pl.ds(base, CHUNK)], idxd_v)
        pltpu.sync_copy(radv_hbm.at[pl.ds(base, CHUNK), :], rv_v)
        pltpu.sync_copy(lg_hbm.at[pl.ds(base, CHUNK), :], lg_v)
        pltpu.async_copy(xkvv_hbm.at[idxs_v], xv_v, sem).wait()

        def edge_body(e, c2):
            lrow = lg_v[e, :]
            ex = jnp.exp(lrow - gm)
            ex = jnp.where(lane < H, ex, 0.0)
            w_v[e, pl.ds(C_V, LP)] = ex
            for h in range(H):
                ex_s = jnp.sum(jnp.where(lane == h, ex, 0.0))
                xv = xv_v[e, pl.ds(h * LP, LP)]
                rv = rv_v[e, pl.ds(h * LP, LP)]
                w_v[e, pl.ds(h * LP, LP)] = xv * rv * ex_s
            return c2

        lax.fori_loop(0, CHUNK, edge_body, 0)
        pltpu.sync_copy(w_v, acc_sh.at[idxd_v], add=True)
        return carry

    lax.fori_loop(0, N_CHUNKS, chunk_body, 0)
    plsc.subcore_barrier()
    pltpu.sync_copy(acc_sh.at[pl.ds(sid * N_PER_TILE, N_PER_TILE), :],
                    u_hbm.at[cid, pl.ds(sid * N_PER_TILE, N_PER_TILE), :])


def kernel(x0, edge_feat, edge_index, W_r1, b_r1, W_r2, W_kv, W_q, W_node,
           W_edge):
    f32 = jnp.float32
    x0_2d = x0[:, :, 0]
    ef = edge_feat[:, :, 0]
    src = edge_index[0]
    dst = edge_index[1]
    b_r1_2d = b_r1[None, :]
    W_node_z = W_node[:C_V]
    W_node_x = W_node[C_V:]
    W_edge_e = W_edge[:C_EDGE]
    W_edge_l = W_edge[C_EDGE:]

    NB = 1000
    xkv_k, xkv_v, qs, x0wn = pl.pallas_call(
        _node_pre_body,
        grid=(N // NB,),
        in_specs=[
            pl.BlockSpec((NB, C_IN), lambda i: (i, 0)),
            pl.BlockSpec((C_IN, C_V + C_KQ), lambda i: (0, 0)),
            pl.BlockSpec((C_IN, C_KQ), lambda i: (0, 0)),
            pl.BlockSpec((C_IN, C_OUT), lambda i: (0, 0)),
        ],
        out_specs=[
            pl.BlockSpec((NB, C_KQ), lambda i: (i, 0)),
            pl.BlockSpec((NB, C_V), lambda i: (i, 0)),
            pl.BlockSpec((NB, C_KQ), lambda i: (i, 0)),
            pl.BlockSpec((NB, C_OUT), lambda i: (i, 0)),
        ],
        out_shape=[
            jax.ShapeDtypeStruct((N, C_KQ), f32),
            jax.ShapeDtypeStruct((N, C_V), f32),
            jax.ShapeDtypeStruct((N, C_KQ), f32),
            jax.ShapeDtypeStruct((N, C_OUT), f32),
        ],
    )(x0_2d, W_kv, W_q, W_node_x)

    EB = 4000
    rad_k, rad_v, ebase = pl.pallas_call(
        _edge_pre_body,
        grid=(E // EB,),
        in_specs=[
            pl.BlockSpec((EB, C_EDGE), lambda i: (i, 0)),
            pl.BlockSpec((C_EDGE, R_HID), lambda i: (0, 0)),
            pl.BlockSpec((1, R_HID), lambda i: (0, 0)),
            pl.BlockSpec((R_HID, C_V + C_KQ), lambda i: (0, 0)),
            pl.BlockSpec((C_EDGE, C_EDGE), lambda i: (0, 0)),
        ],
        out_specs=[
            pl.BlockSpec((EB, C_KQ), lambda i: (i, 0)),
            pl.BlockSpec((EB, C_V), lambda i: (i, 0)),
            pl.BlockSpec((EB, C_EDGE), lambda i: (i, 0)),
        ],
        out_shape=[
            jax.ShapeDtypeStruct((E, C_KQ), f32),
            jax.ShapeDtypeStruct((E, C_V), f32),
            jax.ShapeDtypeStruct((E, C_EDGE), f32),
        ],
    )(ef, W_r1, b_r1_2d, W_r2, W_edge_e)

    mesh = plsc.VectorSubcoreMesh(core_axis_name="c", subcore_axis_name="s")

    sc_a = pl.kernel(
        _sc_logits_body,
        out_type=(
            jax.ShapeDtypeStruct((E, LP), f32),
            jax.ShapeDtypeStruct((NW, LP), f32),
        ),
        mesh=mesh,
        scratch_types=[
            pltpu.VMEM((CHUNK,), jnp.int32),
            pltpu.VMEM((CHUNK,), jnp.int32),
            pltpu.VMEM((CHUNK, C_KQ), f32),
            pltpu.VMEM((CHUNK, C_KQ), f32),
            pltpu.VMEM((CHUNK, C_KQ), f32),
            pltpu.VMEM((CHUNK, LP), f32),
            pltpu.VMEM((LP,), f32),
            pltpu.SemaphoreType.DMA,
        ],
        compiler_params=pltpu.CompilerParams(needs_layout_passes=False, use_tc_tiling_on_sc=False),
    )
    logits16, tmax = sc_a(src, dst, xkv_k, qs, rad_k)

    zeros_acc = jnp.zeros((N, ACC_W), f32)
    sc_b = pl.kernel(
        _sc_scatter_body,
        out_type=jax.ShapeDtypeStruct((NC, N, ACC_W), f32),
        mesh=mesh,
        scratch_types=[
            pltpu.VMEM((CHUNK,), jnp.int32),
            pltpu.VMEM((CHUNK,), jnp.int32),
            pltpu.VMEM((CHUNK, C_V), f32),
            pltpu.VMEM((CHUNK, C_V), f32),
            pltpu.VMEM((CHUNK, LP), f32),
            pltpu.VMEM((CHUNK, ACC_W), f32),
            pltpu.VMEM((NW, LP), f32),
            pltpu.VMEM_SHARED((N, ACC_W), f32),
            pltpu.SemaphoreType.DMA,
        ],
        compiler_params=pltpu.CompilerParams(needs_layout_passes=False, use_tc_tiling_on_sc=False),
    )
    u2 = sc_b(src, dst, logits16, xkv_v, rad_v, tmax, zeros_acc)

    node_out = pl.pallas_call(
        _node_out_body,
        grid=(N // NB,),
        in_specs=[
            pl.BlockSpec((NC, NB, ACC_W), lambda i: (0, i, 0)),
            pl.BlockSpec((NB, C_OUT), lambda i: (i, 0)),
            pl.BlockSpec((C_V, C_OUT), lambda i: (0, 0)),
        ],
        out_specs=pl.BlockSpec((NB, C_OUT), lambda i: (i, 0)),
        out_shape=jax.ShapeDtypeStruct((N, C_OUT), f32),
    )(u2, x0wn, W_node_z)

    edge_out = pl.pallas_call(
        _edge_out_body,
        grid=(E // EB,),
        in_specs=[
            pl.BlockSpec((EB, C_EDGE), lambda i: (i, 0)),
            pl.BlockSpec((EB, LP), lambda i: (i, 0)),
            pl.BlockSpec((H, C_EDGE), lambda i: (0, 0)),
        ],
        out_specs=pl.BlockSpec((EB, C_EDGE), lambda i: (i, 0)),
        out_shape=jax.ShapeDtypeStruct((E, C_EDGE), f32),
    )(ebase, logits16, W_edge_l)

    return (node_out[:, :, None], edge_out[:, :, None])


# speedup vs baseline: 29.9704x; 1.5738x over previous
import functools

import jax
import jax.numpy as jnp
from jax import lax
from jax.experimental import pallas as pl
from jax.experimental.pallas import tpu as pltpu
from jax.experimental.pallas import tpu_sc as plsc

N = 10000
E = 320000
C_IN = 128
C_EDGE = 17
H = 4
C_KQ = 64
C_V = 64
C_OUT = 128
R_HID = 32

NC = 2
NS = 16
NW = NC * NS
LP = 16
CHUNK = 80
E_PER_TILE = E // NW
N_CHUNKS = E_PER_TILE // CHUNK
N_PER_TILE = N // NS
ACC_W = 80


def _node_pre_body(x0_ref, wkv_ref, wq_ref, wnx_ref,
                   xkvk_ref, xkvv_ref, qs_ref, x0wn_ref):
    x = x0_ref[...]
    kv = jnp.dot(x, wkv_ref[...], preferred_element_type=jnp.float32)
    xkvv_ref[...] = kv[:, :C_V]
    xkvk_ref[...] = kv[:, C_V:]
    qs_ref[...] = jnp.dot(x, wq_ref[...], preferred_element_type=jnp.float32) * 0.125
    x0wn_ref[...] = jnp.dot(x, wnx_ref[...], preferred_element_type=jnp.float32)


def _edge_pre_body(ef_ref, wr1_ref, br1_ref, wr2_ref, wee_ref,
                   radk_ref, radv_ref, ebase_ref):
    ef = ef_ref[...]
    h = jnp.maximum(jnp.dot(ef, wr1_ref[...], preferred_element_type=jnp.float32)
                    + br1_ref[...], 0.0)
    rad = jnp.dot(h, wr2_ref[...], preferred_element_type=jnp.float32)
    radv_ref[...] = rad[:, :C_V]
    radk_ref[...] = rad[:, C_V:]
    ebase_ref[...] = jnp.dot(ef, wee_ref[...], preferred_element_type=jnp.float32)


def _edge_out_body(ebase_ref, lg_ref, wel_ref, eout_ref):
    lg = lg_ref[...][:, :H]
    eout_ref[...] = ebase_ref[...] + jnp.dot(
        lg, wel_ref[...], preferred_element_type=jnp.float32)


def _node_out_body(u2_ref, x0wn_ref, wnz_ref, nout_ref):
    u = u2_ref[0] + u2_ref[1]
    w = u[:, :C_V]
    s4 = u[:, C_V:C_V + H]
    hh = lax.broadcasted_iota(jnp.int32, (H, C_V), 0)
    ll = lax.broadcasted_iota(jnp.int32, (H, C_V), 1) // (C_V // H)
    rep = (hh == ll).astype(jnp.float32)
    srep = jnp.dot(s4, rep, preferred_element_type=jnp.float32)
    z = w / jnp.maximum(srep, 1e-30)
    nout_ref[...] = jnp.dot(z, wnz_ref[...], preferred_element_type=jnp.float32) \
        + x0wn_ref[...]


def _sc_logits_body(src_hbm, dst_hbm, xkvk_hbm, qs_hbm, radk_hbm,
                    lg_hbm, tmax_hbm,
                    idxs0, idxs1, idxd0, idxd1, xk0, xk1, q0, q1,
                    rk0, rk1, lg0, lg1, m_v,
                    si0, si1, sg0, sg1, so0, so1):
    cid = lax.axis_index("c")
    sid = lax.axis_index("s")
    wid = sid * NC + cid
    tile_base = wid * E_PER_TILE

    idxs = [idxs0, idxs1]
    idxd = [idxd0, idxd1]
    xk = [xk0, xk1]
    q = [q0, q1]
    rk = [rk0, rk1]
    lg = [lg0, lg1]
    si = [si0, si1]
    sg = [sg0, sg1]
    so = [so0, so1]

    lane = lax.iota(jnp.int32, LP)

    def l1(j, p):
        base = tile_base + j * CHUNK
        pltpu.async_copy(src_hbm.at[pl.ds(base, CHUNK)], idxs[p], si[p])
        pltpu.async_copy(dst_hbm.at[pl.ds(base, CHUNK)], idxd[p], si[p])

    def wait_l1(p):
        pltpu.make_async_copy(src_hbm.at[pl.ds(0, CHUNK)], idxs[p], si[p]).wait()
        pltpu.make_async_copy(dst_hbm.at[pl.ds(0, CHUNK)], idxd[p], si[p]).wait()

    def l2(j, b, p):
        base = tile_base + j * CHUNK
        pltpu.async_copy(radk_hbm.at[pl.ds(base, CHUNK), :], rk[b], sg[b])
        pltpu.async_copy(xkvk_hbm.at[idxs[p]], xk[b], sg[b])
        pltpu.async_copy(qs_hbm.at[idxd[p]], q[b], sg[b])

    def wait_l2(b, p):
        pltpu.make_async_copy(radk_hbm.at[pl.ds(0, CHUNK), :], rk[b], sg[b]).wait()
        pltpu.make_async_copy(xkvk_hbm.at[idxs[p]], xk[b], sg[b]).wait()
        pltpu.make_async_copy(qs_hbm.at[idxd[p]], q[b], sg[b]).wait()

    def out(j, b):
        base = tile_base + j * CHUNK
        pltpu.async_copy(lg[b], lg_hbm.at[pl.ds(base, CHUNK), :], so[b])

    def wait_out(b):
        pltpu.make_async_copy(lg[b], lg_hbm.at[pl.ds(0, CHUNK), :], so[b]).wait()

    def compute(j, b, m_carry):
        xkb, rkb, qb, lgb = xk[b], rk[b], q[b], lg[b]

        def edge_body(e, m_in):
            m_out = m_in
            srow = jnp.zeros((LP,), jnp.float32)
            for h in range(H):
                a = xkb[e, pl.ds(h * LP, LP)]
                bb = rkb[e, pl.ds(h * LP, LP)]
                c = qb[e, pl.ds(h * LP, LP)]
                s = jnp.sum(a * bb * c)
                srow = jnp.where(lane == h, s, srow)
                m_out = jnp.maximum(m_out, s)
            lgb[e, :] = srow
            return m_out

        return lax.fori_loop(0, CHUNK, edge_body, m_carry)

    l1(0, 0)
    l1(1, 1)
    wait_l1(0)
    l2(0, 0, 0)

    def pair(t, m_carry):
        m_c = m_carry
        for b in (0, 1):
            j = 2 * t + b
            bn = b ^ 1
            wait_l1(bn)
            l2(j + 1, bn, bn)
            wait_l2(b, b)

            @pl.when(j >= 2)
            def _():
                wait_out(b)

            m_c = compute(j, b, m_c)
            out(j, b)

            @pl.when(j + 2 < N_CHUNKS)
            def _():
                l1(j + 2, b)
        return m_c

    m = lax.fori_loop(0, (N_CHUNKS - 1) // 2, pair, jnp.float32(-3.0e38))
    wait_l2(0, 0)
    wait_out(0)
    m = compute(N_CHUNKS - 1, 0, m)
    out(N_CHUNKS - 1, 0)
    wait_out(1)
    wait_out(0)
    m_v[...] = jnp.full((LP,), m, dtype=jnp.float32)
    pltpu.sync_copy(m_v, tmax_hbm.at[wid])


def _sc_scatter_body(src_hbm, dst_hbm, lg_hbm, xkvv_hbm, radv_hbm,
                     tmax_hbm, zeros_hbm,
                     u_hbm,
                     idxs0, idxs1, idxs2, idxs3, idxd0, idxd1, idxd2, idxd3,
                     xv0, xv1, rv0, rv1, lb0, lb1, w0, w1, tm_v, acc_sh,
                     si0, si1, si2, si3, sg0, sg1, ss0, ss1):
    cid = lax.axis_index("c")
    sid = lax.axis_index("s")
    wid = sid * NC + cid
    tile_base = wid * E_PER_TILE

    idxs = [idxs0, idxs1, idxs2, idxs3]
    idxd = [idxd0, idxd1, idxd2, idxd3]
    xv = [xv0, xv1]
    rv = [rv0, rv1]
    lb = [lb0, lb1]
    w = [w0, w1]
    si = [si0, si1, si2, si3]
    sg = [sg0, sg1]
    ss = [ss0, ss1]

    pltpu.sync_copy(tmax_hbm, tm_v)

    def max_body(i, m_in):
        return jnp.maximum(m_in, jnp.max(tm_v[i]))

    gm = lax.fori_loop(0, NW, max_body, jnp.float32(-3.0e38))

    pltpu.sync_copy(zeros_hbm.at[pl.ds(sid * N_PER_TILE, N_PER_TILE), :],
                    acc_sh.at[pl.ds(sid * N_PER_TILE, N_PER_TILE), :])
    plsc.subcore_barrier()

    lane = lax.iota(jnp.int32, LP)

    def l1(j, p):
        base = tile_base + j * CHUNK
        pltpu.async_copy(src_hbm.at[pl.ds(base, CHUNK)], idxs[p], si[p])
        pltpu.async_copy(dst_hbm.at[pl.ds(base, CHUNK)], idxd[p], si[p])

    def wait_l1(p):
        pltpu.make_async_copy(src_hbm.at[pl.ds(0, CHUNK)], idxs[p], si[p]).wait()
        pltpu.make_async_copy(dst_hbm.at[pl.ds(0, CHUNK)], idxd[p], si[p]).wait()

    def l2(j, b, p):
        base = tile_base + j * CHUNK
        pltpu.async_copy(radv_hbm.at[pl.ds(base, CHUNK), :], rv[b], sg[b])
        pltpu.async_copy(lg_hbm.at[pl.ds(base, CHUNK), :], lb[b], sg[b])
        pltpu.async_copy(xkvv_hbm.at[idxs[p]], xv[b], sg[b])

    def wait_l2(b, p):
        pltpu.make_async_copy(radv_hbm.at[pl.ds(0, CHUNK), :], rv[b], sg[b]).wait()
        pltpu.make_async_copy(lg_hbm.at[pl.ds(0, CHUNK), :], lb[b], sg[b]).wait()
        pltpu.make_async_copy(xkvv_hbm.at[idxs[p]], xv[b], sg[b]).wait()

    def scat(j, b, p):
        pltpu.async_copy(w[b], acc_sh.at[idxd[p]], ss[b], add=True)

    def wait_scat(b, p):
        pltpu.make_async_copy(w[b], acc_sh.at[idxd[p]], ss[b]).wait()

    def compute(j, b):
        xvb, rvb, lbb, wb = xv[b], rv[b], lb[b], w[b]

        def edge_body(e, c2):
            lrow = lbb[e, :]
            ex = jnp.exp(lrow - gm)
            ex = jnp.where(lane < H, ex, 0.0)
            wb[e, pl.ds(C_V, LP)] = ex
            for h in range(H):
                ex_s = jnp.sum(jnp.where(lane == h, ex, 0.0))
                xvv = xvb[e, pl.ds(h * LP, LP)]
                rvv = rvb[e, pl.ds(h * LP, LP)]
                wb[e, pl.ds(h * LP, LP)] = xvv * rvv * ex_s
            return c2

        lax.fori_loop(0, CHUNK, edge_body, 0)

    l1(0, 0)
    l1(1, 1)
    wait_l1(0)
    l2(0, 0, 0)

    def quad(t, carry):
        for b4 in range(4):
            j = 4 * t + b4
            b = b4 % 2
            p = b4
            pn = (b4 + 1) % 4
            p2 = (b4 + 2) % 4
            wait_l1(pn)
            l2(j + 1, b ^ 1, pn)
            wait_l2(b, p)

            @pl.when(j >= 2)
            def _():
                wait_scat(b, p2)

            compute(j, b)
            scat(j, b, p)

            @pl.when(j + 2 < N_CHUNKS)
            def _():
                l1(j + 2, p2)
        return carry

    lax.fori_loop(0, (N_CHUNKS - 1) // 4, quad, 0)
    wait_l2(0, 0)
    wait_scat(0, 2)
    compute(N_CHUNKS - 1, 0)
    scat(N_CHUNKS - 1, 0, 0)
    wait_scat(1, 3)
    wait_scat(0, 0)
    plsc.subcore_barrier()
    pltpu.sync_copy(acc_sh.at[pl.ds(sid * N_PER_TILE, N_PER_TILE), :],
                    u_hbm.at[cid, pl.ds(sid * N_PER_TILE, N_PER_TILE), :])


def kernel(x0, edge_feat, edge_index, W_r1, b_r1, W_r2, W_kv, W_q, W_node,
           W_edge):
    f32 = jnp.float32
    x0_2d = x0[:, :, 0]
    ef = edge_feat[:, :, 0]
    src = edge_index[0]
    dst = edge_index[1]
    b_r1_2d = b_r1[None, :]
    W_node_z = W_node[:C_V]
    W_node_x = W_node[C_V:]
    W_edge_e = W_edge[:C_EDGE]
    W_edge_l = W_edge[C_EDGE:]

    NB = 1000
    xkv_k, xkv_v, qs, x0wn = pl.pallas_call(
        _node_pre_body,
        grid=(N // NB,),
        in_specs=[
            pl.BlockSpec((NB, C_IN), lambda i: (i, 0)),
            pl.BlockSpec((C_IN, C_V + C_KQ), lambda i: (0, 0)),
            pl.BlockSpec((C_IN, C_KQ), lambda i: (0, 0)),
            pl.BlockSpec((C_IN, C_OUT), lambda i: (0, 0)),
        ],
        out_specs=[
            pl.BlockSpec((NB, C_KQ), lambda i: (i, 0)),
            pl.BlockSpec((NB, C_V), lambda i: (i, 0)),
            pl.BlockSpec((NB, C_KQ), lambda i: (i, 0)),
            pl.BlockSpec((NB, C_OUT), lambda i: (i, 0)),
        ],
        out_shape=[
            jax.ShapeDtypeStruct((N, C_KQ), f32),
            jax.ShapeDtypeStruct((N, C_V), f32),
            jax.ShapeDtypeStruct((N, C_KQ), f32),
            jax.ShapeDtypeStruct((N, C_OUT), f32),
        ],
    )(x0_2d, W_kv, W_q, W_node_x)

    EB = 4000
    rad_k, rad_v, ebase = pl.pallas_call(
        _edge_pre_body,
        grid=(E // EB,),
        in_specs=[
            pl.BlockSpec((EB, C_EDGE), lambda i: (i, 0)),
            pl.BlockSpec((C_EDGE, R_HID), lambda i: (0, 0)),
            pl.BlockSpec((1, R_HID), lambda i: (0, 0)),
            pl.BlockSpec((R_HID, C_V + C_KQ), lambda i: (0, 0)),
            pl.BlockSpec((C_EDGE, C_EDGE), lambda i: (0, 0)),
        ],
        out_specs=[
            pl.BlockSpec((EB, C_KQ), lambda i: (i, 0)),
            pl.BlockSpec((EB, C_V), lambda i: (i, 0)),
            pl.BlockSpec((EB, C_EDGE), lambda i: (i, 0)),
        ],
        out_shape=[
            jax.ShapeDtypeStruct((E, C_KQ), f32),
            jax.ShapeDtypeStruct((E, C_V), f32),
            jax.ShapeDtypeStruct((E, C_EDGE), f32),
        ],
    )(ef, W_r1, b_r1_2d, W_r2, W_edge_e)

    mesh = plsc.VectorSubcoreMesh(core_axis_name="c", subcore_axis_name="s")

    sc_a = pl.kernel(
        _sc_logits_body,
        out_type=(
            jax.ShapeDtypeStruct((E, LP), f32),
            jax.ShapeDtypeStruct((NW, LP), f32),
        ),
        mesh=mesh,
        scratch_types=(
            [pltpu.VMEM((CHUNK,), jnp.int32)] * 4
            + [pltpu.VMEM((CHUNK, C_KQ), f32)] * 6
            + [pltpu.VMEM((CHUNK, LP), f32)] * 2
            + [pltpu.VMEM((LP,), f32)]
            + [pltpu.SemaphoreType.DMA] * 6
        ),
        compiler_params=pltpu.CompilerParams(needs_layout_passes=False, use_tc_tiling_on_sc=False),
    )
    logits16, tmax = sc_a(src, dst, xkv_k, qs, rad_k)

    zeros_acc = jnp.zeros((N, ACC_W), f32)
    sc_b = pl.kernel(
        _sc_scatter_body,
        out_type=jax.ShapeDtypeStruct((NC, N, ACC_W), f32),
        mesh=mesh,
        scratch_types=(
            [pltpu.VMEM((CHUNK,), jnp.int32)] * 8
            + [pltpu.VMEM((CHUNK, C_V), f32)] * 4
            + [pltpu.VMEM((CHUNK, LP), f32)] * 2
            + [pltpu.VMEM((CHUNK, ACC_W), f32)] * 2
            + [pltpu.VMEM((NW, LP), f32)]
            + [pltpu.VMEM_SHARED((N, ACC_W), f32)]
            + [pltpu.SemaphoreType.DMA] * 8
        ),
        compiler_params=pltpu.CompilerParams(needs_layout_passes=False, use_tc_tiling_on_sc=False),
    )
    u2 = sc_b(src, dst, logits16, xkv_v, rad_v, tmax, zeros_acc)

    node_out = pl.pallas_call(
        _node_out_body,
        grid=(N // NB,),
        in_specs=[
            pl.BlockSpec((NC, NB, ACC_W), lambda i: (0, i, 0)),
            pl.BlockSpec((NB, C_OUT), lambda i: (i, 0)),
            pl.BlockSpec((C_V, C_OUT), lambda i: (0, 0)),
        ],
        out_specs=pl.BlockSpec((NB, C_OUT), lambda i: (i, 0)),
        out_shape=jax.ShapeDtypeStruct((N, C_OUT), f32),
    )(u2, x0wn, W_node_z)

    edge_out = pl.pallas_call(
        _edge_out_body,
        grid=(E // EB,),
        in_specs=[
            pl.BlockSpec((EB, C_EDGE), lambda i: (i, 0)),
            pl.BlockSpec((EB, LP), lambda i: (i, 0)),
            pl.BlockSpec((H, C_EDGE), lambda i: (0, 0)),
        ],
        out_specs=pl.BlockSpec((EB, C_EDGE), lambda i: (i, 0)),
        out_shape=jax.ShapeDtypeStruct((E, C_EDGE), f32),
    )(ebase, logits16, W_edge_l)

    return (node_out[:, :, None], edge_out[:, :, None])


# speedup vs baseline: 30.0209x; 1.0017x over previous
import functools

import jax
import jax.numpy as jnp
from jax import lax
from jax.experimental import pallas as pl
from jax.experimental.pallas import tpu as pltpu
from jax.experimental.pallas import tpu_sc as plsc

N = 10000
E = 320000
C_IN = 128
C_EDGE = 17
H = 4
C_KQ = 64
C_V = 64
C_OUT = 128
R_HID = 32

NC = 2
NS = 16
NW = NC * NS
LP = 16
CHUNK = 80
E_PER_TILE = E // NW
N_CHUNKS = E_PER_TILE // CHUNK
N_PER_TILE = N // NS
ACC_W = 68


def _node_pre_body(x0_ref, wkv_ref, wq_ref, wnx_ref,
                   xkvk_ref, xkvv_ref, qs_ref, x0wn_ref):
    x = x0_ref[...]
    kv = jnp.dot(x, wkv_ref[...], preferred_element_type=jnp.float32)
    xkvv_ref[...] = kv[:, :C_V]
    xkvk_ref[...] = kv[:, C_V:]
    qs_ref[...] = jnp.dot(x, wq_ref[...], preferred_element_type=jnp.float32) * 0.125
    x0wn_ref[...] = jnp.dot(x, wnx_ref[...], preferred_element_type=jnp.float32)


def _edge_pre_body(ef_ref, wr1_ref, br1_ref, wr2_ref, wee_ref,
                   radk_ref, radv_ref, ebase_ref):
    ef = ef_ref[...]
    h = jnp.maximum(jnp.dot(ef, wr1_ref[...], preferred_element_type=jnp.float32)
                    + br1_ref[...], 0.0)
    rad = jnp.dot(h, wr2_ref[...], preferred_element_type=jnp.float32)
    radv_ref[...] = rad[:, :C_V]
    radk_ref[...] = rad[:, C_V:]
    ebase_ref[...] = jnp.dot(ef, wee_ref[...], preferred_element_type=jnp.float32)


def _edge_out_body(ebase_ref, lg_ref, wel_ref, eout_ref):
    lg = lg_ref[...][:, :H]
    eout_ref[...] = ebase_ref[...] + jnp.dot(
        lg, wel_ref[...], preferred_element_type=jnp.float32)


def _node_out_body(u2_ref, x0wn_ref, wnz_ref, nout_ref):
    u = u2_ref[0] + u2_ref[1]
    w = u[:, :C_V]
    s4 = u[:, C_V:C_V + H]
    hh = lax.broadcasted_iota(jnp.int32, (H, C_V), 0)
    ll = lax.broadcasted_iota(jnp.int32, (H, C_V), 1) // (C_V // H)
    rep = (hh == ll).astype(jnp.float32)
    srep = jnp.dot(s4, rep, preferred_element_type=jnp.float32)
    z = w / jnp.maximum(srep, 1e-30)
    nout_ref[...] = jnp.dot(z, wnz_ref[...], preferred_element_type=jnp.float32) \
        + x0wn_ref[...]


def _sc_logits_body(src_hbm, dst_hbm, xkvk_hbm, qs_hbm, radk_hbm,
                    lg_hbm, tmax_hbm,
                    idxs0, idxs1, idxd0, idxd1, xk0, xk1, q0, q1,
                    rk0, rk1, lg0, lg1, m_v,
                    si0, si1, sg0, sg1, so0, so1):
    cid = lax.axis_index("c")
    sid = lax.axis_index("s")
    wid = sid * NC + cid
    tile_base = wid * E_PER_TILE

    idxs = [idxs0, idxs1]
    idxd = [idxd0, idxd1]
    xk = [xk0, xk1]
    q = [q0, q1]
    rk = [rk0, rk1]
    lg = [lg0, lg1]
    si = [si0, si1]
    sg = [sg0, sg1]
    so = [so0, so1]

    lane = lax.iota(jnp.int32, LP)

    def l1(j, p):
        base = tile_base + j * CHUNK
        pltpu.async_copy(src_hbm.at[pl.ds(base, CHUNK)], idxs[p], si[p])
        pltpu.async_copy(dst_hbm.at[pl.ds(base, CHUNK)], idxd[p], si[p])

    def wait_l1(p):
        pltpu.make_async_copy(src_hbm.at[pl.ds(0, CHUNK)], idxs[p], si[p]).wait()
        pltpu.make_async_copy(dst_hbm.at[pl.ds(0, CHUNK)], idxd[p], si[p]).wait()

    def l2(j, b, p):
        base = tile_base + j * CHUNK
        pltpu.async_copy(radk_hbm.at[pl.ds(base, CHUNK), :], rk[b], sg[b])
        pltpu.async_copy(xkvk_hbm.at[idxs[p]], xk[b], sg[b])
        pltpu.async_copy(qs_hbm.at[idxd[p]], q[b], sg[b])

    def wait_l2(b, p):
        pltpu.make_async_copy(radk_hbm.at[pl.ds(0, CHUNK), :], rk[b], sg[b]).wait()
        pltpu.make_async_copy(xkvk_hbm.at[idxs[p]], xk[b], sg[b]).wait()
        pltpu.make_async_copy(qs_hbm.at[idxd[p]], q[b], sg[b]).wait()

    def out(j, b):
        base = tile_base + j * CHUNK
        pltpu.async_copy(lg[b], lg_hbm.at[pl.ds(base, CHUNK), :], so[b])

    def wait_out(b):
        pltpu.make_async_copy(lg[b], lg_hbm.at[pl.ds(0, CHUNK), :], so[b]).wait()

    def compute(j, b, m_carry):
        xkb, rkb, qb, lgb = xk[b], rk[b], q[b], lg[b]

        def one_edge(e, m_in):
            m_out = m_in
            srow = jnp.zeros((LP,), jnp.float32)
            for h in range(H):
                a = xkb[e, pl.ds(h * LP, LP)]
                bb = rkb[e, pl.ds(h * LP, LP)]
                c = qb[e, pl.ds(h * LP, LP)]
                s = jnp.sum(a * bb * c)
                srow = jnp.where(lane == h, s, srow)
                m_out = jnp.maximum(m_out, s)
            lgb[e, :] = srow
            return m_out

        def edge_body(e2, m_in):
            m_in = one_edge(2 * e2, m_in)
            return one_edge(2 * e2 + 1, m_in)

        return lax.fori_loop(0, CHUNK // 2, edge_body, m_carry)

    l1(0, 0)
    l1(1, 1)
    wait_l1(0)
    l2(0, 0, 0)

    def pair(t, m_carry):
        m_c = m_carry
        for b in (0, 1):
            j = 2 * t + b
            bn = b ^ 1
            wait_l1(bn)
            l2(j + 1, bn, bn)
            wait_l2(b, b)

            @pl.when(j >= 2)
            def _():
                wait_out(b)

            m_c = compute(j, b, m_c)
            out(j, b)

            @pl.when(j + 2 < N_CHUNKS)
            def _():
                l1(j + 2, b)
        return m_c

    m = lax.fori_loop(0, (N_CHUNKS - 1) // 2, pair, jnp.float32(-3.0e38))
    wait_l2(0, 0)
    wait_out(0)
    m = compute(N_CHUNKS - 1, 0, m)
    out(N_CHUNKS - 1, 0)
    wait_out(1)
    wait_out(0)
    m_v[...] = jnp.full((LP,), m, dtype=jnp.float32)
    pltpu.sync_copy(m_v, tmax_hbm.at[wid])


def _sc_scatter_body(src_hbm, dst_hbm, lg_hbm, xkvv_hbm, radv_hbm,
                     tmax_hbm, zeros_hbm,
                     u_hbm,
                     idxs0, idxs1, idxs2, idxs3, idxd0, idxd1, idxd2, idxd3,
                     xv0, xv1, rv0, rv1, lb0, lb1, w0, w1, tm_v, acc_sh,
                     si0, si1, si2, si3, sg0, sg1, ss0, ss1):
    cid = lax.axis_index("c")
    sid = lax.axis_index("s")
    wid = sid * NC + cid
    tile_base = wid * E_PER_TILE

    idxs = [idxs0, idxs1, idxs2, idxs3]
    idxd = [idxd0, idxd1, idxd2, idxd3]
    xv = [xv0, xv1]
    rv = [rv0, rv1]
    lb = [lb0, lb1]
    w = [w0, w1]
    si = [si0, si1, si2, si3]
    sg = [sg0, sg1]
    ss = [ss0, ss1]

    pltpu.sync_copy(tmax_hbm, tm_v)

    def max_body(i, m_in):
        return jnp.maximum(m_in, jnp.max(tm_v[i]))

    gm = lax.fori_loop(0, NW, max_body, jnp.float32(-3.0e38))

    pltpu.sync_copy(zeros_hbm.at[pl.ds(sid * N_PER_TILE, N_PER_TILE), :],
                    acc_sh.at[pl.ds(sid * N_PER_TILE, N_PER_TILE), :])
    plsc.subcore_barrier()

    lane = lax.iota(jnp.int32, LP)

    def l1(j, p):
        base = tile_base + j * CHUNK
        pltpu.async_copy(src_hbm.at[pl.ds(base, CHUNK)], idxs[p], si[p])
        pltpu.async_copy(dst_hbm.at[pl.ds(base, CHUNK)], idxd[p], si[p])

    def wait_l1(p):
        pltpu.make_async_copy(src_hbm.at[pl.ds(0, CHUNK)], idxs[p], si[p]).wait()
        pltpu.make_async_copy(dst_hbm.at[pl.ds(0, CHUNK)], idxd[p], si[p]).wait()

    def l2(j, b, p):
        base = tile_base + j * CHUNK
        pltpu.async_copy(radv_hbm.at[pl.ds(base, CHUNK), :], rv[b], sg[b])
        pltpu.async_copy(lg_hbm.at[pl.ds(base, CHUNK), :], lb[b], sg[b])
        pltpu.async_copy(xkvv_hbm.at[idxs[p]], xv[b], sg[b])

    def wait_l2(b, p):
        pltpu.make_async_copy(radv_hbm.at[pl.ds(0, CHUNK), :], rv[b], sg[b]).wait()
        pltpu.make_async_copy(lg_hbm.at[pl.ds(0, CHUNK), :], lb[b], sg[b]).wait()
        pltpu.make_async_copy(xkvv_hbm.at[idxs[p]], xv[b], sg[b]).wait()

    def scat(j, b, p):
        pltpu.async_copy(w[b], acc_sh.at[idxd[p]], ss[b], add=True)

    def wait_scat(b, p):
        pltpu.make_async_copy(w[b], acc_sh.at[idxd[p]], ss[b]).wait()

    bidx = [jnp.full((LP,), h, jnp.int32) for h in range(H)]
    shift_idx = (lane - (LP - H)) & (LP - 1)
    shift_msk = lane >= (LP - H)

    def compute(j, b):
        xvb, rvb, lbb, wb = xv[b], rv[b], lb[b], w[b]

        def one_edge(e):
            lrow = lbb[e, :]
            ex = jnp.exp(lrow - gm)
            ex = jnp.where(lane < H, ex, 0.0)
            ex_sh = jnp.where(
                shift_msk,
                ex.at[shift_idx].get(mode="promise_in_bounds"), 0.0)
            wb[e, pl.ds(ACC_W - LP, LP)] = ex_sh
            for h in range(H):
                ex_b = ex.at[bidx[h]].get(mode="promise_in_bounds")
                xvv = xvb[e, pl.ds(h * LP, LP)]
                rvv = rvb[e, pl.ds(h * LP, LP)]
                wb[e, pl.ds(h * LP, LP)] = xvv * rvv * ex_b
            return e

        def edge_body(e2, c2):
            one_edge(2 * e2)
            one_edge(2 * e2 + 1)
            return c2

        lax.fori_loop(0, CHUNK // 2, edge_body, 0)

    l1(0, 0)
    l1(1, 1)
    wait_l1(0)
    l2(0, 0, 0)

    def quad(t, carry):
        for b4 in range(4):
            j = 4 * t + b4
            b = b4 % 2
            p = b4
            pn = (b4 + 1) % 4
            p2 = (b4 + 2) % 4
            wait_l1(pn)
            l2(j + 1, b ^ 1, pn)
            wait_l2(b, p)

            @pl.when(j >= 2)
            def _():
                wait_scat(b, p2)

            compute(j, b)
            scat(j, b, p)

            @pl.when(j + 2 < N_CHUNKS)
            def _():
                l1(j + 2, p2)
        return carry

    lax.fori_loop(0, (N_CHUNKS - 1) // 4, quad, 0)
    wait_l2(0, 0)
    wait_scat(0, 2)
    compute(N_CHUNKS - 1, 0)
    scat(N_CHUNKS - 1, 0, 0)
    wait_scat(1, 3)
    wait_scat(0, 0)
    plsc.subcore_barrier()
    pltpu.sync_copy(acc_sh.at[pl.ds(sid * N_PER_TILE, N_PER_TILE), :],
                    u_hbm.at[cid, pl.ds(sid * N_PER_TILE, N_PER_TILE), :])


def kernel(x0, edge_feat, edge_index, W_r1, b_r1, W_r2, W_kv, W_q, W_node,
           W_edge):
    f32 = jnp.float32
    x0_2d = x0[:, :, 0]
    ef = edge_feat[:, :, 0]
    src = edge_index[0]
    dst = edge_index[1]
    b_r1_2d = b_r1[None, :]
    W_node_z = W_node[:C_V]
    W_node_x = W_node[C_V:]
    W_edge_e = W_edge[:C_EDGE]
    W_edge_l = W_edge[C_EDGE:]

    NB = 1000
    xkv_k, xkv_v, qs, x0wn = pl.pallas_call(
        _node_pre_body,
        grid=(N // NB,),
        in_specs=[
            pl.BlockSpec((NB, C_IN), lambda i: (i, 0)),
            pl.BlockSpec((C_IN, C_V + C_KQ), lambda i: (0, 0)),
            pl.BlockSpec((C_IN, C_KQ), lambda i: (0, 0)),
            pl.BlockSpec((C_IN, C_OUT), lambda i: (0, 0)),
        ],
        out_specs=[
            pl.BlockSpec((NB, C_KQ), lambda i: (i, 0)),
            pl.BlockSpec((NB, C_V), lambda i: (i, 0)),
            pl.BlockSpec((NB, C_KQ), lambda i: (i, 0)),
            pl.BlockSpec((NB, C_OUT), lambda i: (i, 0)),
        ],
        out_shape=[
            jax.ShapeDtypeStruct((N, C_KQ), f32),
            jax.ShapeDtypeStruct((N, C_V), f32),
            jax.ShapeDtypeStruct((N, C_KQ), f32),
            jax.ShapeDtypeStruct((N, C_OUT), f32),
        ],
    )(x0_2d, W_kv, W_q, W_node_x)

    EB = 4000
    rad_k, rad_v, ebase = pl.pallas_call(
        _edge_pre_body,
        grid=(E // EB,),
        in_specs=[
            pl.BlockSpec((EB, C_EDGE), lambda i: (i, 0)),
            pl.BlockSpec((C_EDGE, R_HID), lambda i: (0, 0)),
            pl.BlockSpec((1, R_HID), lambda i: (0, 0)),
            pl.BlockSpec((R_HID, C_V + C_KQ), lambda i: (0, 0)),
            pl.BlockSpec((C_EDGE, C_EDGE), lambda i: (0, 0)),
        ],
        out_specs=[
            pl.BlockSpec((EB, C_KQ), lambda i: (i, 0)),
            pl.BlockSpec((EB, C_V), lambda i: (i, 0)),
            pl.BlockSpec((EB, C_EDGE), lambda i: (i, 0)),
        ],
        out_shape=[
            jax.ShapeDtypeStruct((E, C_KQ), f32),
            jax.ShapeDtypeStruct((E, C_V), f32),
            jax.ShapeDtypeStruct((E, C_EDGE), f32),
        ],
    )(ef, W_r1, b_r1_2d, W_r2, W_edge_e)

    mesh = plsc.VectorSubcoreMesh(core_axis_name="c", subcore_axis_name="s")

    sc_a = pl.kernel(
        _sc_logits_body,
        out_type=(
            jax.ShapeDtypeStruct((E, LP), f32),
            jax.ShapeDtypeStruct((NW, LP), f32),
        ),
        mesh=mesh,
        scratch_types=(
            [pltpu.VMEM((CHUNK,), jnp.int32)] * 4
            + [pltpu.VMEM((CHUNK, C_KQ), f32)] * 6
            + [pltpu.VMEM((CHUNK, LP), f32)] * 2
            + [pltpu.VMEM((LP,), f32)]
            + [pltpu.SemaphoreType.DMA] * 6
        ),
        compiler_params=pltpu.CompilerParams(needs_layout_passes=False, use_tc_tiling_on_sc=False),
    )
    logits16, tmax = sc_a(src, dst, xkv_k, qs, rad_k)

    zeros_acc = jnp.zeros((N, ACC_W), f32)
    sc_b = pl.kernel(
        _sc_scatter_body,
        out_type=jax.ShapeDtypeStruct((NC, N, ACC_W), f32),
        mesh=mesh,
        scratch_types=(
            [pltpu.VMEM((CHUNK,), jnp.int32)] * 8
            + [pltpu.VMEM((CHUNK, C_V), f32)] * 4
            + [pltpu.VMEM((CHUNK, LP), f32)] * 2
            + [pltpu.VMEM((CHUNK, ACC_W), f32)] * 2
            + [pltpu.VMEM((NW, LP), f32)]
            + [pltpu.VMEM_SHARED((N, ACC_W), f32)]
            + [pltpu.SemaphoreType.DMA] * 8
        ),
        compiler_params=pltpu.CompilerParams(needs_layout_passes=False, use_tc_tiling_on_sc=False),
    )
    u2 = sc_b(src, dst, logits16, xkv_v, rad_v, tmax, zeros_acc)

    node_out = pl.pallas_call(
        _node_out_body,
        grid=(N // NB,),
        in_specs=[
            pl.BlockSpec((NC, NB, ACC_W), lambda i: (0, i, 0)),
            pl.BlockSpec((NB, C_OUT), lambda i: (i, 0)),
            pl.BlockSpec((C_V, C_OUT), lambda i: (0, 0)),
        ],
        out_specs=pl.BlockSpec((NB, C_OUT), lambda i: (i, 0)),
        out_shape=jax.ShapeDtypeStruct((N, C_OUT), f32),
    )(u2, x0wn, W_node_z)

    edge_out = pl.pallas_call(
        _edge_out_body,
        grid=(E // EB,),
        in_specs=[
            pl.BlockSpec((EB, C_EDGE), lambda i: (i, 0)),
            pl.BlockSpec((EB, LP), lambda i: (i, 0)),
            pl.BlockSpec((H, C_EDGE), lambda i: (0, 0)),
        ],
        out_specs=pl.BlockSpec((EB, C_EDGE), lambda i: (i, 0)),
        out_shape=jax.ShapeDtypeStruct((E, C_EDGE), f32),
    )(ebase, logits16, W_edge_l)

    return (node_out[:, :, None], edge_out[:, :, None])


# speedup vs baseline: 30.3572x; 1.0112x over previous
import functools

import jax
import jax.numpy as jnp
from jax import lax
from jax.experimental import pallas as pl
from jax.experimental.pallas import tpu as pltpu
from jax.experimental.pallas import tpu_sc as plsc

N = 10000
E = 320000
C_IN = 128
C_EDGE = 17
H = 4
C_KQ = 64
C_V = 64
C_OUT = 128
R_HID = 32

NC = 2
NS = 16
NW = NC * NS
LP = 16
CHUNK = 80
E_PER_TILE = E // NW
N_CHUNKS = E_PER_TILE // CHUNK
N_PER_TILE = N // NS
ACC_W = 80


def _node_pre_body(x0_ref, wkv_ref, wq_ref, wnx_ref,
                   xkvk_ref, xkvv_ref, qs_ref, x0wn_ref):
    x = x0_ref[...]
    kv = jnp.dot(x, wkv_ref[...], preferred_element_type=jnp.float32)
    xkvv_ref[...] = kv[:, :C_V]
    xkvk_ref[...] = kv[:, C_V:]
    qs_ref[...] = jnp.dot(x, wq_ref[...], preferred_element_type=jnp.float32) * 0.125
    x0wn_ref[...] = jnp.dot(x, wnx_ref[...], preferred_element_type=jnp.float32)


def _edge_pre_body(ef_ref, wr1_ref, br1_ref, wr2_ref, wee_ref,
                   radk_ref, radv_ref, ebase_ref):
    ef = ef_ref[...]
    h = jnp.maximum(jnp.dot(ef, wr1_ref[...], preferred_element_type=jnp.float32)
                    + br1_ref[...], 0.0)
    rad = jnp.dot(h, wr2_ref[...], preferred_element_type=jnp.float32)
    radv_ref[...] = rad[:, :C_V]
    radk_ref[...] = rad[:, C_V:]
    ebase_ref[...] = jnp.dot(ef, wee_ref[...], preferred_element_type=jnp.float32)


def _edge_out_body(ebase_ref, lg_ref, wel_ref, eout_ref):
    lg = lg_ref[...][:, :H]
    eout_ref[...] = ebase_ref[...] + jnp.dot(
        lg, wel_ref[...], preferred_element_type=jnp.float32)


def _node_out_body(u2_ref, x0wn_ref, wnz_ref, nout_ref):
    u = u2_ref[0] + u2_ref[1]
    w = u[:, :C_V]
    s4 = u[:, C_V:C_V + H]
    hh = lax.broadcasted_iota(jnp.int32, (H, C_V), 0)
    ll = lax.broadcasted_iota(jnp.int32, (H, C_V), 1) // (C_V // H)
    rep = (hh == ll).astype(jnp.float32)
    srep = jnp.dot(s4, rep, preferred_element_type=jnp.float32)
    z = w / jnp.maximum(srep, 1e-30)
    nout_ref[...] = jnp.dot(z, wnz_ref[...], preferred_element_type=jnp.float32) \
        + x0wn_ref[...]


def _sc_logits_body(src_hbm, dst_hbm, xkvk_hbm, qs_hbm, radk_hbm,
                    lg_hbm, tmax_hbm,
                    idxs0, idxs1, idxd0, idxd1, xk0, xk1, q0, q1,
                    rk0, rk1, lg0, lg1, m_v,
                    si0, si1, sg0, sg1, so0, so1):
    cid = lax.axis_index("c")
    sid = lax.axis_index("s")
    wid = sid * NC + cid
    tile_base = wid * E_PER_TILE

    idxs = [idxs0, idxs1]
    idxd = [idxd0, idxd1]
    xk = [xk0, xk1]
    q = [q0, q1]
    rk = [rk0, rk1]
    lg = [lg0, lg1]
    si = [si0, si1]
    sg = [sg0, sg1]
    so = [so0, so1]

    lane = lax.iota(jnp.int32, LP)

    def l1(j, p):
        base = tile_base + j * CHUNK
        pltpu.async_copy(src_hbm.at[pl.ds(base, CHUNK)], idxs[p], si[p])
        pltpu.async_copy(dst_hbm.at[pl.ds(base, CHUNK)], idxd[p], si[p])

    def wait_l1(p):
        pltpu.make_async_copy(src_hbm.at[pl.ds(0, CHUNK)], idxs[p], si[p]).wait()
        pltpu.make_async_copy(dst_hbm.at[pl.ds(0, CHUNK)], idxd[p], si[p]).wait()

    def l2(j, b, p):
        base = tile_base + j * CHUNK
        pltpu.async_copy(radk_hbm.at[pl.ds(base, CHUNK), :], rk[b], sg[b])
        pltpu.async_copy(xkvk_hbm.at[idxs[p]], xk[b], sg[b])
        pltpu.async_copy(qs_hbm.at[idxd[p]], q[b], sg[b])

    def wait_l2(b, p):
        pltpu.make_async_copy(radk_hbm.at[pl.ds(0, CHUNK), :], rk[b], sg[b]).wait()
        pltpu.make_async_copy(xkvk_hbm.at[idxs[p]], xk[b], sg[b]).wait()
        pltpu.make_async_copy(qs_hbm.at[idxd[p]], q[b], sg[b]).wait()

    def out(j, b):
        base = tile_base + j * CHUNK
        pltpu.async_copy(lg[b], lg_hbm.at[pl.ds(base, CHUNK), :], so[b])

    def wait_out(b):
        pltpu.make_async_copy(lg[b], lg_hbm.at[pl.ds(0, CHUNK), :], so[b]).wait()

    def compute(j, b, m_carry):
        xkb, rkb, qb, lgb = xk[b], rk[b], q[b], lg[b]

        def one_edge(e, m_in):
            m_out = m_in
            srow = jnp.zeros((LP,), jnp.float32)
            for h in range(H):
                a = xkb[e, pl.ds(h * LP, LP)]
                bb = rkb[e, pl.ds(h * LP, LP)]
                c = qb[e, pl.ds(h * LP, LP)]
                s = jnp.sum(a * bb * c)
                srow = jnp.where(lane == h, s, srow)
                m_out = jnp.maximum(m_out, s)
            lgb[e, :] = srow
            return m_out

        def edge_body(e2, m_in):
            m_in = one_edge(2 * e2, m_in)
            return one_edge(2 * e2 + 1, m_in)

        return lax.fori_loop(0, CHUNK // 2, edge_body, m_carry)

    l1(0, 0)
    l1(1, 1)
    wait_l1(0)
    l2(0, 0, 0)

    def pair(t, m_carry):
        m_c = m_carry
        for b in (0, 1):
            j = 2 * t + b
            bn = b ^ 1
            wait_l1(bn)
            l2(j + 1, bn, bn)
            wait_l2(b, b)

            @pl.when(j >= 2)
            def _():
                wait_out(b)

            m_c = compute(j, b, m_c)
            out(j, b)

            @pl.when(j + 2 < N_CHUNKS)
            def _():
                l1(j + 2, b)
        return m_c

    m = lax.fori_loop(0, (N_CHUNKS - 1) // 2, pair, jnp.float32(-3.0e38))
    wait_l2(0, 0)
    wait_out(0)
    m = compute(N_CHUNKS - 1, 0, m)
    out(N_CHUNKS - 1, 0)
    wait_out(1)
    wait_out(0)
    m_v[...] = jnp.full((LP,), m, dtype=jnp.float32)
    pltpu.sync_copy(m_v, tmax_hbm.at[wid])


def _sc_scatter_body(src_hbm, dst_hbm, lg_hbm, xkvv_hbm, radv_hbm,
                     tmax_hbm, zeros_hbm,
                     u_hbm,
                     idxs0, idxs1, idxs2, idxs3, idxd0, idxd1, idxd2, idxd3,
                     xv0, xv1, rv0, rv1, lb0, lb1, w0, w1, tm_v, acc_sh,
                     si0, si1, si2, si3, sg0, sg1, ss0, ss1):
    cid = lax.axis_index("c")
    sid = lax.axis_index("s")
    wid = sid * NC + cid
    tile_base = wid * E_PER_TILE

    idxs = [idxs0, idxs1, idxs2, idxs3]
    idxd = [idxd0, idxd1, idxd2, idxd3]
    xv = [xv0, xv1]
    rv = [rv0, rv1]
    lb = [lb0, lb1]
    w = [w0, w1]
    si = [si0, si1, si2, si3]
    sg = [sg0, sg1]
    ss = [ss0, ss1]

    pltpu.sync_copy(tmax_hbm, tm_v)

    def max_body(i, m_in):
        return jnp.maximum(m_in, jnp.max(tm_v[i]))

    gm = lax.fori_loop(0, NW, max_body, jnp.float32(-3.0e38))

    pltpu.sync_copy(zeros_hbm.at[pl.ds(sid * N_PER_TILE, N_PER_TILE), :],
                    acc_sh.at[pl.ds(sid * N_PER_TILE, N_PER_TILE), :])
    plsc.subcore_barrier()

    lane = lax.iota(jnp.int32, LP)

    def l1(j, p):
        base = tile_base + j * CHUNK
        pltpu.async_copy(src_hbm.at[pl.ds(base, CHUNK)], idxs[p], si[p])
        pltpu.async_copy(dst_hbm.at[pl.ds(base, CHUNK)], idxd[p], si[p])

    def wait_l1(p):
        pltpu.make_async_copy(src_hbm.at[pl.ds(0, CHUNK)], idxs[p], si[p]).wait()
        pltpu.make_async_copy(dst_hbm.at[pl.ds(0, CHUNK)], idxd[p], si[p]).wait()

    def l2(j, b, p):
        base = tile_base + j * CHUNK
        pltpu.async_copy(radv_hbm.at[pl.ds(base, CHUNK), :], rv[b], sg[b])
        pltpu.async_copy(lg_hbm.at[pl.ds(base, CHUNK), :], lb[b], sg[b])
        pltpu.async_copy(xkvv_hbm.at[idxs[p]], xv[b], sg[b])

    def wait_l2(b, p):
        pltpu.make_async_copy(radv_hbm.at[pl.ds(0, CHUNK), :], rv[b], sg[b]).wait()
        pltpu.make_async_copy(lg_hbm.at[pl.ds(0, CHUNK), :], lb[b], sg[b]).wait()
        pltpu.make_async_copy(xkvv_hbm.at[idxs[p]], xv[b], sg[b]).wait()

    def scat(j, b, p):
        pltpu.async_copy(w[b], acc_sh.at[idxd[p]], ss[b], add=True)

    def wait_scat(b, p):
        pltpu.make_async_copy(w[b], acc_sh.at[idxd[p]], ss[b]).wait()

    bidx = [jnp.full((LP,), h, jnp.int32) for h in range(H)]

    def compute(j, b):
        xvb, rvb, lbb, wb = xv[b], rv[b], lb[b], w[b]

        def one_edge(e):
            lrow = lbb[e, :]
            ex = jnp.exp(lrow - gm)
            ex = jnp.where(lane < H, ex, 0.0)
            wb[e, pl.ds(C_V, LP)] = ex
            for h in range(H):
                ex_b = ex.at[bidx[h]].get(mode="promise_in_bounds")
                xvv = xvb[e, pl.ds(h * LP, LP)]
                rvv = rvb[e, pl.ds(h * LP, LP)]
                wb[e, pl.ds(h * LP, LP)] = xvv * rvv * ex_b
            return e

        def edge_body(e2, c2):
            one_edge(2 * e2)
            one_edge(2 * e2 + 1)
            return c2

        lax.fori_loop(0, CHUNK // 2, edge_body, 0)

    l1(0, 0)
    l1(1, 1)
    wait_l1(0)
    l2(0, 0, 0)

    def quad(t, carry):
        for b4 in range(4):
            j = 4 * t + b4
            b = b4 % 2
            p = b4
            pn = (b4 + 1) % 4
            p2 = (b4 + 2) % 4
            wait_l1(pn)
            l2(j + 1, b ^ 1, pn)
            wait_l2(b, p)

            @pl.when(j >= 2)
            def _():
                wait_scat(b, p2)

            compute(j, b)
            scat(j, b, p)

            @pl.when(j + 2 < N_CHUNKS)
            def _():
                l1(j + 2, p2)
        return carry

    lax.fori_loop(0, (N_CHUNKS - 1) // 4, quad, 0)
    wait_l2(0, 0)
    wait_scat(0, 2)
    compute(N_CHUNKS - 1, 0)
    scat(N_CHUNKS - 1, 0, 0)
    wait_scat(1, 3)
    wait_scat(0, 0)
    plsc.subcore_barrier()
    pltpu.sync_copy(acc_sh.at[pl.ds(sid * N_PER_TILE, N_PER_TILE), :],
                    u_hbm.at[cid, pl.ds(sid * N_PER_TILE, N_PER_TILE), :])


def kernel(x0, edge_feat, edge_index, W_r1, b_r1, W_r2, W_kv, W_q, W_node,
           W_edge):
    f32 = jnp.float32
    x0_2d = x0[:, :, 0]
    ef = edge_feat[:, :, 0]
    src = edge_index[0]
    dst = edge_index[1]
    b_r1_2d = b_r1[None, :]
    W_node_z = W_node[:C_V]
    W_node_x = W_node[C_V:]
    W_edge_e = W_edge[:C_EDGE]
    W_edge_l = W_edge[C_EDGE:]

    NB = 1000
    xkv_k, xkv_v, qs, x0wn = pl.pallas_call(
        _node_pre_body,
        grid=(N // NB,),
        in_specs=[
            pl.BlockSpec((NB, C_IN), lambda i: (i, 0)),
            pl.BlockSpec((C_IN, C_V + C_KQ), lambda i: (0, 0)),
            pl.BlockSpec((C_IN, C_KQ), lambda i: (0, 0)),
            pl.BlockSpec((C_IN, C_OUT), lambda i: (0, 0)),
        ],
        out_specs=[
            pl.BlockSpec((NB, C_KQ), lambda i: (i, 0)),
            pl.BlockSpec((NB, C_V), lambda i: (i, 0)),
            pl.BlockSpec((NB, C_KQ), lambda i: (i, 0)),
            pl.BlockSpec((NB, C_OUT), lambda i: (i, 0)),
        ],
        out_shape=[
            jax.ShapeDtypeStruct((N, C_KQ), f32),
            jax.ShapeDtypeStruct((N, C_V), f32),
            jax.ShapeDtypeStruct((N, C_KQ), f32),
            jax.ShapeDtypeStruct((N, C_OUT), f32),
        ],
    )(x0_2d, W_kv, W_q, W_node_x)

    EB = 4000
    rad_k, rad_v, ebase = pl.pallas_call(
        _edge_pre_body,
        grid=(E // EB,),
        in_specs=[
            pl.BlockSpec((EB, C_EDGE), lambda i: (i, 0)),
            pl.BlockSpec((C_EDGE, R_HID), lambda i: (0, 0)),
            pl.BlockSpec((1, R_HID), lambda i: (0, 0)),
            pl.BlockSpec((R_HID, C_V + C_KQ), lambda i: (0, 0)),
            pl.BlockSpec((C_EDGE, C_EDGE), lambda i: (0, 0)),
        ],
        out_specs=[
            pl.BlockSpec((EB, C_KQ), lambda i: (i, 0)),
            pl.BlockSpec((EB, C_V), lambda i: (i, 0)),
            pl.BlockSpec((EB, C_EDGE), lambda i: (i, 0)),
        ],
        out_shape=[
            jax.ShapeDtypeStruct((E, C_KQ), f32),
            jax.ShapeDtypeStruct((E, C_V), f32),
            jax.ShapeDtypeStruct((E, C_EDGE), f32),
        ],
    )(ef, W_r1, b_r1_2d, W_r2, W_edge_e)

    mesh = plsc.VectorSubcoreMesh(core_axis_name="c", subcore_axis_name="s")

    sc_a = pl.kernel(
        _sc_logits_body,
        out_type=(
            jax.ShapeDtypeStruct((E, LP), f32),
            jax.ShapeDtypeStruct((NW, LP), f32),
        ),
        mesh=mesh,
        scratch_types=(
            [pltpu.VMEM((CHUNK,), jnp.int32)] * 4
            + [pltpu.VMEM((CHUNK, C_KQ), f32)] * 6
            + [pltpu.VMEM((CHUNK, LP), f32)] * 2
            + [pltpu.VMEM((LP,), f32)]
            + [pltpu.SemaphoreType.DMA] * 6
        ),
        compiler_params=pltpu.CompilerParams(needs_layout_passes=False, use_tc_tiling_on_sc=False),
    )
    logits16, tmax = sc_a(src, dst, xkv_k, qs, rad_k)

    zeros_acc = jnp.zeros((N, ACC_W), f32)
    sc_b = pl.kernel(
        _sc_scatter_body,
        out_type=jax.ShapeDtypeStruct((NC, N, ACC_W), f32),
        mesh=mesh,
        scratch_types=(
            [pltpu.VMEM((CHUNK,), jnp.int32)] * 8
            + [pltpu.VMEM((CHUNK, C_V), f32)] * 4
            + [pltpu.VMEM((CHUNK, LP), f32)] * 2
            + [pltpu.VMEM((CHUNK, ACC_W), f32)] * 2
            + [pltpu.VMEM((NW, LP), f32)]
            + [pltpu.VMEM_SHARED((N, ACC_W), f32)]
            + [pltpu.SemaphoreType.DMA] * 8
        ),
        compiler_params=pltpu.CompilerParams(needs_layout_passes=False, use_tc_tiling_on_sc=False),
    )
    u2 = sc_b(src, dst, logits16, xkv_v, rad_v, tmax, zeros_acc)

    node_out = pl.pallas_call(
        _node_out_body,
        grid=(N // NB,),
        in_specs=[
            pl.BlockSpec((NC, NB, ACC_W), lambda i: (0, i, 0)),
            pl.BlockSpec((NB, C_OUT), lambda i: (i, 0)),
            pl.BlockSpec((C_V, C_OUT), lambda i: (0, 0)),
        ],
        out_specs=pl.BlockSpec((NB, C_OUT), lambda i: (i, 0)),
        out_shape=jax.ShapeDtypeStruct((N, C_OUT), f32),
    )(u2, x0wn, W_node_z)

    edge_out = pl.pallas_call(
        _edge_out_body,
        grid=(E // EB,),
        in_specs=[
            pl.BlockSpec((EB, C_EDGE), lambda i: (i, 0)),
            pl.BlockSpec((EB, LP), lambda i: (i, 0)),
            pl.BlockSpec((H, C_EDGE), lambda i: (0, 0)),
        ],
        out_specs=pl.BlockSpec((EB, C_EDGE), lambda i: (i, 0)),
        out_shape=jax.ShapeDtypeStruct((E, C_EDGE), f32),
    )(ebase, logits16, W_edge_l)

    return (node_out[:, :, None], edge_out[:, :, None])
